# Initial kernel scaffold; baseline (speedup 1.0000x reference)
import jax
import jax.numpy as jnp
from jax.experimental import pallas as pl


def kernel(preds, targets):
    raise NotImplementedError("write your pallas kernel here")


# speedup vs baseline: 27.0980x; 27.0980x over previous
import functools

import jax
import jax.numpy as jnp
from jax import lax
from jax.experimental import pallas as pl
from jax.experimental.pallas import tpu as pltpu
from jax.experimental.pallas import tpu_sc as plsc

N = 128
NC, NS, L = 2, 16, 16
NW = NC * NS
ROWS_PER_W = (2 * N) // NW

_mesh = plsc.VectorSubcoreMesh(core_axis_name="c", subcore_axis_name="s")
_params = pltpu.CompilerParams(needs_layout_passes=False)


def _wid():
    return lax.axis_index("s") * NC + lax.axis_index("c")


@functools.partial(
    pl.kernel,
    out_type=jax.ShapeDtypeStruct((2 * N * N,), jnp.float32),
    mesh=_mesh,
    compiler_params=_params,
    scratch_types=[
        pltpu.VMEM((ROWS_PER_W * N,), jnp.float32),
        pltpu.VMEM((2 * ROWS_PER_W * N,), jnp.float32),
        pltpu.VMEM((ROWS_PER_W * N,), jnp.float32),
    ],
)
def _row_dt_kernel(masks_hbm, g2_hbm, m_v, dbuf_v, out_v):
    wid = _wid()
    base = wid * (ROWS_PER_W * N)
    pltpu.sync_copy(masks_hbm.at[pl.ds(base, ROWS_PER_W * N)], m_v)

    lane = lax.iota(jnp.int32, L)
    rowoff = (lane & 7) * N
    sletf = lane < 8

    def sweep(j, d):
        col = jnp.where(sletf, j, (N - 1) - j)
        m = plsc.load_gather(m_v, [rowoff + col])
        d = jnp.where(m > 0.5, jnp.float32(0.0), d + 1.0)
        plsc.store_scatter(dbuf_v, [lane * N + col], d)
        return d

    lax.fori_loop(0, N, sweep, jnp.full((L,), jnp.inf, jnp.float32))

    for r in range(ROWS_PER_W):
        for c in range(N // L):
            off = r * N + c * L
            dl = dbuf_v[pl.ds(off, L)]
            dr = dbuf_v[pl.ds(ROWS_PER_W * N + off, L)]
            mn = jnp.minimum(dl, dr)
            out_v[pl.ds(off, L)] = mn * mn

    pltpu.sync_copy(out_v, g2_hbm.at[pl.ds(base, ROWS_PER_W * N)])


I_PER_W = N // 2
IB = 8


@functools.partial(
    pl.kernel,
    out_type=jax.ShapeDtypeStruct((NW * L,), jnp.float32),
    mesh=_mesh,
    compiler_params=_params,
    scratch_types=[
        pltpu.VMEM((N * N,), jnp.float32),
        pltpu.VMEM((I_PER_W * N,), jnp.float32),
        pltpu.VMEM((L,), jnp.float32),
    ],
)
def _colpass_kernel(masks_hbm, g2_hbm, out_hbm, g2_v, ma_v, best_v):
    wid = _wid()
    d = wid // 16
    sub = wid % 16
    jc = (sub % 8) * L
    i0 = (sub // 8) * I_PER_W

    pltpu.sync_copy(g2_hbm.at[pl.ds((1 - d) * N * N, N * N)], g2_v)
    pltpu.sync_copy(masks_hbm.at[pl.ds(d * N * N + i0 * N, I_PER_W * N)], ma_v)

    best = jnp.full((L,), -jnp.inf, jnp.float32)
    inf16 = jnp.full((L,), jnp.inf, jnp.float32)

    for ib in range(I_PER_W // IB):

        def minplus(i2, accs):
            g = g2_v[pl.ds(i2 * N + jc, L)]
            out = []
            for k in range(IB):
                di = (i0 + ib * IB + k) - i2
                out.append(jnp.minimum(accs[k], g + (di * di).astype(jnp.float32)))
            return tuple(out)

        accs = lax.fori_loop(0, N, minplus, (inf16,) * IB)

        for k in range(IB):
            m = ma_v[pl.ds((ib * IB + k) * N + jc, L)]
            best = jnp.maximum(best, jnp.where(m > 0.5, accs[k], -jnp.inf))

    best_v[...] = best
    pltpu.sync_copy(best_v, out_hbm.at[pl.ds(wid * L, L)])


def kernel(preds, targets):
    masks = jnp.concatenate([preds.reshape(-1), targets.reshape(-1)])
    g2 = _row_dt_kernel(masks)
    partials = _colpass_kernel(masks, g2)
    max_min = jnp.max(partials.reshape(2, 16 * L), axis=1)
    hd = jnp.sqrt(max_min)
    return jnp.maximum(hd[0], hd[1])


# speedup vs baseline: 30.5471x; 1.1273x over previous
import functools

import jax
import jax.numpy as jnp
from jax import lax
from jax.experimental import pallas as pl
from jax.experimental.pallas import tpu as pltpu
from jax.experimental.pallas import tpu_sc as plsc

N = 128
L = 16
RPW = 8
IHALF = N // 2
IB = 8

_mesh = plsc.VectorSubcoreMesh(core_axis_name="c", subcore_axis_name="s")
_params = pltpu.CompilerParams(needs_layout_passes=False)


@functools.partial(
    pl.kernel,
    out_type=jax.ShapeDtypeStruct((2 * 16 * L,), jnp.float32),
    mesh=_mesh,
    compiler_params=_params,
    scratch_types=[
        pltpu.VMEM((RPW * N,), jnp.float32),
        pltpu.VMEM((2 * RPW * N,), jnp.float32),
        pltpu.VMEM((RPW * N,), jnp.float32),
        pltpu.VMEM_SHARED((N * N,), jnp.float32),
        pltpu.VMEM((N * N,), jnp.float32),
        pltpu.VMEM((IHALF * N,), jnp.float32),
        pltpu.VMEM((L,), jnp.float32),
    ],
)
def _hausdorff_kernel(masks_hbm, out_hbm, m_v, dbuf_v, g2row_v, g2_sh, g2_v,
                      ma_v, best_v):
    d = lax.axis_index("c")
    s = lax.axis_index("s")

    base = s * (RPW * N)
    pltpu.sync_copy(masks_hbm.at[pl.ds((1 - d) * N * N + base, RPW * N)], m_v)

    lane = lax.iota(jnp.int32, L)
    rowoff = (lane & 7) * N
    fwd = lane < 8

    def sweep(j, dist):
        col = jnp.where(fwd, j, (N - 1) - j)
        m = plsc.load_gather(m_v, [rowoff + col])
        dist = jnp.where(m > 0.5, jnp.float32(0.0), dist + 1.0)
        plsc.store_scatter(dbuf_v, [lane * N + col], dist)
        return dist

    lax.fori_loop(0, N, sweep, jnp.full((L,), jnp.inf, jnp.float32))

    for r in range(RPW):
        for c in range(N // L):
            off = r * N + c * L
            mn = jnp.minimum(dbuf_v[pl.ds(off, L)],
                             dbuf_v[pl.ds(RPW * N + off, L)])
            g2row_v[pl.ds(off, L)] = mn * mn

    pltpu.sync_copy(g2row_v, g2_sh.at[pl.ds(base, RPW * N)])
    plsc.subcore_barrier()
    pltpu.sync_copy(g2_sh, g2_v)

    jc = (s % 8) * L
    i0 = (s // 8) * IHALF
    pltpu.sync_copy(masks_hbm.at[pl.ds(d * N * N + i0 * N, IHALF * N)], ma_v)

    best = jnp.full((L,), -jnp.inf, jnp.float32)
    inf16 = jnp.full((L,), jnp.inf, jnp.float32)

    for ib in range(IHALF // IB):

        def minplus(i2, accs):
            g = g2_v[pl.ds(i2 * N + jc, L)]
            out = []
            for k in range(IB):
                di = (i0 + ib * IB + k) - i2
                out.append(jnp.minimum(accs[k], g + (di * di).astype(jnp.float32)))
            return tuple(out)

        accs = lax.fori_loop(0, N, minplus, (inf16,) * IB)

        for k in range(IB):
            m = ma_v[pl.ds((ib * IB + k) * N + jc, L)]
            best = jnp.maximum(best, jnp.where(m > 0.5, accs[k], -jnp.inf))

    best_v[...] = best
    pltpu.sync_copy(best_v, out_hbm.at[pl.ds((d * 16 + s) * L, L)])


def kernel(preds, targets):
    masks = jnp.concatenate([preds.reshape(-1), targets.reshape(-1)])
    partials = _hausdorff_kernel(masks)
    max_min = jnp.max(partials.reshape(2, 16 * L), axis=1)
    hd = jnp.sqrt(max_min)
    return jnp.maximum(hd[0], hd[1])


# speedup vs baseline: 33.5420x; 1.0980x over previous
import functools

import jax
import jax.numpy as jnp
from jax import lax
from jax.experimental import pallas as pl
from jax.experimental.pallas import tpu as pltpu
from jax.experimental.pallas import tpu_sc as plsc

N = 128
L = 16
RPW = 8
IHALF = N // 2
IB = 8
CS = 8

_mesh = plsc.VectorSubcoreMesh(core_axis_name="c", subcore_axis_name="s")
_params = pltpu.CompilerParams(needs_layout_passes=False)


@functools.partial(
    pl.kernel,
    out_type=jax.ShapeDtypeStruct((2 * 16 * L,), jnp.float32),
    mesh=_mesh,
    compiler_params=_params,
    scratch_types=[
        pltpu.VMEM((RPW, N), jnp.float32),
        pltpu.VMEM((2 * RPW * N,), jnp.float32),
        pltpu.VMEM((RPW * N,), jnp.float32),
        pltpu.VMEM_SHARED((N * N,), jnp.float32),
        pltpu.VMEM_SHARED((16 * L,), jnp.float32),
        pltpu.VMEM((N * N,), jnp.float32),
        pltpu.VMEM((IHALF, N), jnp.float32),
        pltpu.VMEM((16 * L,), jnp.float32),
        pltpu.VMEM((L,), jnp.float32),
    ],
)
def _hausdorff_kernel(preds_hbm, targets_hbm, out_hbm, m_v, dbuf_v, g2row_v,
                      g2_sh, u_sh, g2_v, ma_v, u_v, stage_v):
    d = lax.axis_index("c")
    s = lax.axis_index("s")

    @pl.when(d == 0)
    def _():
        pltpu.sync_copy(targets_hbm.at[pl.ds(s * RPW, RPW), :], m_v)

    @pl.when(d == 1)
    def _():
        pltpu.sync_copy(preds_hbm.at[pl.ds(s * RPW, RPW), :], m_v)

    lane = lax.iota(jnp.int32, L)
    row_l = lane & 7
    fwd = lane < 8

    def sweep(j, dist):
        col = jnp.where(fwd, j, (N - 1) - j)
        m = plsc.load_gather(m_v, [row_l, col])
        dist = jnp.where(m > 0.5, jnp.float32(0.0), dist + 1.0)
        plsc.store_scatter(dbuf_v, [lane * N + col], dist)
        return dist

    lax.fori_loop(0, N, sweep, jnp.full((L,), jnp.inf, jnp.float32))

    for r in range(RPW):
        for c in range(N // L):
            off = r * N + c * L
            mn = jnp.minimum(dbuf_v[pl.ds(off, L)],
                             dbuf_v[pl.ds(RPW * N + off, L)])
            g2row_v[pl.ds(off, L)] = mn * mn

    pltpu.sync_copy(g2row_v, g2_sh.at[pl.ds(s * RPW * N, RPW * N)])
    plsc.subcore_barrier()
    pltpu.sync_copy(g2_sh, g2_v)

    jc = (s % 8) * L
    i0 = (s // 8) * IHALF

    @pl.when(d == 0)
    def _():
        pltpu.sync_copy(preds_hbm.at[pl.ds(i0, IHALF), :], ma_v)

    @pl.when(d == 1)
    def _():
        pltpu.sync_copy(targets_hbm.at[pl.ds(i0, IHALF), :], ma_v)

    inf16 = jnp.full((L,), jnp.inf, jnp.float32)

    def masked_fold(best, accs, ib):
        for k in range(IB):
            m = ma_v[ib * IB + k, pl.ds(jc, L)]
            best = jnp.maximum(best, jnp.where(m > 0.5, accs[k], -jnp.inf))
        return best

    ub = jnp.full((L,), -jnp.inf, jnp.float32)
    for ib in range(IHALF // IB):

        def coarse(t, accs):
            i2 = t * CS
            g = g2_v[pl.ds(i2 * N + jc, L)]
            out = []
            for k in range(IB):
                di = (i0 + ib * IB + k) - i2
                out.append(jnp.minimum(accs[k], g + (di * di).astype(jnp.float32)))
            return tuple(out)

        accs = lax.fori_loop(0, N // CS, coarse, (inf16,) * IB)
        ub = masked_fold(ub, accs, ib)

    stage_v[...] = ub
    pltpu.sync_copy(stage_v, u_sh.at[pl.ds(s * L, L)])
    plsc.subcore_barrier()
    pltpu.sync_copy(u_sh, u_v)

    bv = u_v[pl.ds(0, L)]
    for c in range(1, 16):
        bv = jnp.maximum(bv, u_v[pl.ds(c * L, L)])
    bound = jnp.max(bv)

    radius = jnp.int32(0)
    for c in range(N // L):
        dv = (lane + (c * L + 1)).astype(jnp.float32)
        radius = radius + jnp.sum(jnp.where(dv * dv <= bound, 1, 0))

    best = jnp.full((L,), -jnp.inf, jnp.float32)
    for ib in range(IHALF // IB):
        base_i = i0 + ib * IB
        lo = jnp.maximum(base_i - radius, 0)
        hi = jnp.minimum(base_i + (IB - 1) + radius + 1, N)

        def minplus(i2, accs):
            g = g2_v[pl.ds(i2 * N + jc, L)]
            out = []
            for k in range(IB):
                di = (i0 + ib * IB + k) - i2
                out.append(jnp.minimum(accs[k], g + (di * di).astype(jnp.float32)))
            return tuple(out)

        accs = lax.fori_loop(lo, hi, minplus, (inf16,) * IB)
        best = masked_fold(best, accs, ib)

    stage_v[...] = best
    pltpu.sync_copy(stage_v, out_hbm.at[pl.ds((d * 16 + s) * L, L)])


def kernel(preds, targets):
    partials = _hausdorff_kernel(preds, targets)
    max_min = jnp.max(partials.reshape(2, 16 * L), axis=1)
    hd = jnp.sqrt(max_min)
    return jnp.maximum(hd[0], hd[1])


# speedup vs baseline: 33.8137x; 1.0081x over previous
import functools

import jax
import jax.numpy as jnp
from jax import lax
from jax.experimental import pallas as pl
from jax.experimental.pallas import tpu as pltpu
from jax.experimental.pallas import tpu_sc as plsc

N = 128
L = 16
RPW = 8
IHALF = N // 2
IB = 8

_mesh = plsc.VectorSubcoreMesh(core_axis_name="c", subcore_axis_name="s")
_params = pltpu.CompilerParams(needs_layout_passes=False)


@functools.partial(
    pl.kernel,
    out_type=jax.ShapeDtypeStruct((2 * 16 * L,), jnp.float32),
    mesh=_mesh,
    compiler_params=_params,
    scratch_types=[
        pltpu.VMEM((RPW, N), jnp.float32),
        pltpu.VMEM((2 * RPW * N,), jnp.float32),
        pltpu.VMEM((8, RPW, L), jnp.float32),
        pltpu.VMEM_SHARED((8, N, L), jnp.float32),
        pltpu.VMEM_SHARED((16 * L,), jnp.float32),
        pltpu.VMEM((N, L), jnp.float32),
        pltpu.VMEM((IHALF, N), jnp.float32),
        pltpu.VMEM((16 * L,), jnp.float32),
        pltpu.VMEM((L,), jnp.float32),
    ],
)
def _hausdorff_kernel(preds_hbm, targets_hbm, out_hbm, m_v, dbuf_v, g2row_v,
                      g2_sh, u_sh, g2c_v, ma_v, u_v, stage_v):
    d = lax.axis_index("c")
    s = lax.axis_index("s")

    @pl.when(d == 0)
    def _():
        pltpu.sync_copy(targets_hbm.at[pl.ds(s * RPW, RPW), :], m_v)

    @pl.when(d == 1)
    def _():
        pltpu.sync_copy(preds_hbm.at[pl.ds(s * RPW, RPW), :], m_v)

    lane = lax.iota(jnp.int32, L)
    row_l = lane & 7
    fwd = lane < 8

    def sweep(j, dist):
        col = jnp.where(fwd, j, (N - 1) - j)
        m = plsc.load_gather(m_v, [row_l, col])
        dist = jnp.where(m > 0.5, jnp.float32(0.0), dist + 1.0)
        plsc.store_scatter(dbuf_v, [lane * N + col], dist)
        return dist

    lax.fori_loop(0, N, sweep, jnp.full((L,), jnp.inf, jnp.float32))

    for r in range(RPW):
        for c in range(N // L):
            off = r * N + c * L
            mn = jnp.minimum(dbuf_v[pl.ds(off, L)],
                             dbuf_v[pl.ds(RPW * N + off, L)])
            g2row_v[c, r, :] = mn * mn

    for c in range(N // L):
        pltpu.sync_copy(g2row_v.at[c], g2_sh.at[c].at[pl.ds(s * RPW, RPW), :])
    plsc.subcore_barrier()

    q = s % 8
    jc = q * L
    i0 = (s // 8) * IHALF
    pltpu.sync_copy(g2_sh.at[q], g2c_v)

    @pl.when(d == 0)
    def _():
        pltpu.sync_copy(preds_hbm.at[pl.ds(i0, IHALF), :], ma_v)

    @pl.when(d == 1)
    def _():
        pltpu.sync_copy(targets_hbm.at[pl.ds(i0, IHALF), :], ma_v)

    ub = jnp.full((L,), -jnp.inf, jnp.float32)
    for row in range(IHALF):
        m = ma_v[row, pl.ds(jc, L)]
        g = g2c_v[i0 + row, :]
        ub = jnp.maximum(ub, jnp.where(m > 0.5, g, -jnp.inf))

    stage_v[...] = ub
    pltpu.sync_copy(stage_v, u_sh.at[pl.ds(s * L, L)])
    plsc.subcore_barrier()
    pltpu.sync_copy(u_sh, u_v)

    bv = u_v[pl.ds(0, L)]
    for c in range(1, 16):
        bv = jnp.maximum(bv, u_v[pl.ds(c * L, L)])
    bound = jnp.max(bv)

    radius = jnp.int32(0)
    for c in range(N // L):
        dv = (lane + (c * L + 1)).astype(jnp.float32)
        radius = radius + jnp.sum(jnp.where(dv * dv <= bound, 1, 0))

    inf16 = jnp.full((L,), jnp.inf, jnp.float32)
    best = jnp.full((L,), -jnp.inf, jnp.float32)
    for ib in range(IHALF // IB):
        base_i = i0 + ib * IB
        lo = jnp.maximum(base_i - radius, 0)
        hi = jnp.minimum(base_i + (IB - 1) + radius + 1, N)

        def minplus(i2, accs):
            g = g2c_v[i2, :]
            out = []
            for k in range(IB):
                di = (i0 + ib * IB + k) - i2
                out.append(jnp.minimum(accs[k], g + (di * di).astype(jnp.float32)))
            return tuple(out)

        accs = lax.fori_loop(lo, hi, minplus, (inf16,) * IB)

        for k in range(IB):
            m = ma_v[ib * IB + k, pl.ds(jc, L)]
            best = jnp.maximum(best, jnp.where(m > 0.5, accs[k], -jnp.inf))

    stage_v[...] = best
    pltpu.sync_copy(stage_v, out_hbm.at[pl.ds((d * 16 + s) * L, L)])


def kernel(preds, targets):
    partials = _hausdorff_kernel(preds, targets)
    max_min = jnp.max(partials.reshape(2, 16 * L), axis=1)
    hd = jnp.sqrt(max_min)
    return jnp.maximum(hd[0], hd[1])
